# Initial kernel scaffold; baseline (speedup 1.0000x reference)
import jax
import jax.numpy as jnp
from jax.experimental import pallas as pl


def kernel(x, weight):
    raise NotImplementedError("write your pallas kernel here")


# speedup vs baseline: 2.9716x; 2.9716x over previous
import functools

import jax
import jax.numpy as jnp
from jax import lax
from jax.experimental import pallas as pl
from jax.experimental.pallas import tpu as pltpu
from jax.experimental.pallas import tpu_sc as plsc

_D = 128
_NC = 2
_NS = 16
_NW = _NC * _NS
_CHUNK = 128


@functools.lru_cache(maxsize=None)
def _make_gather(n_idx: int, n_rows: int):
    n_per_w = n_idx // _NW
    n_chunks = n_per_w // _CHUNK

    mesh = plsc.VectorSubcoreMesh(core_axis_name="c", subcore_axis_name="s")

    @functools.partial(
        pl.kernel,
        mesh=mesh,
        out_type=jax.ShapeDtypeStruct((n_idx, _D), jnp.float32),
        scratch_types=[
            pltpu.VMEM((n_chunks, _CHUNK), jnp.int32),
            pltpu.VMEM((_CHUNK, _D), jnp.float32),
            pltpu.SemaphoreType.DMA,
        ],
    )
    def gather(idx_hbm, table_hbm, out_hbm, idx_v, rows_v, g_sem):
        wid = lax.axis_index("s") * _NC + lax.axis_index("c")
        base = wid * n_per_w
        pltpu.sync_copy(idx_hbm.at[wid], idx_v)

        def body(i, carry):
            pltpu.async_copy(table_hbm.at[idx_v.at[i]], rows_v, g_sem).wait()
            pltpu.sync_copy(rows_v, out_hbm.at[pl.ds(base + i * _CHUNK, _CHUNK)])
            return carry

        lax.fori_loop(0, n_chunks, body, 0)

    return gather


def kernel(x, weight):
    flat = x.reshape(-1).astype(jnp.int32)
    idx3 = flat.reshape(_NW, -1, _CHUNK)
    out = _make_gather(flat.size, weight.shape[0])(idx3, weight)
    return out.reshape(x.shape + (weight.shape[-1],))


# speedup vs baseline: 3.3448x; 1.1256x over previous
import functools

import jax
import jax.numpy as jnp
from jax import lax
from jax.experimental import pallas as pl
from jax.experimental.pallas import tpu as pltpu
from jax.experimental.pallas import tpu_sc as plsc

_D = 128
_NC = 2
_NS = 16
_NW = _NC * _NS
_CHUNK = 128
_NBUF = 5
_AHEAD = 2


@functools.lru_cache(maxsize=None)
def _make_gather(n_idx: int, n_rows: int):
    n_per_w = n_idx // _NW
    n_chunks = n_per_w // _CHUNK
    n_groups = n_chunks // _NBUF

    mesh = plsc.VectorSubcoreMesh(core_axis_name="c", subcore_axis_name="s")

    @functools.partial(
        pl.kernel,
        mesh=mesh,
        out_type=jax.ShapeDtypeStruct((n_idx, _D), jnp.float32),
        scratch_types=[
            pltpu.VMEM((n_chunks, _CHUNK), jnp.int32),
            pltpu.VMEM((_NBUF, _CHUNK, _D), jnp.float32),
            pltpu.SemaphoreType.DMA((_NBUF,)),
            pltpu.SemaphoreType.DMA((_NBUF,)),
        ],
    )
    def gather(idx_hbm, table_hbm, out_hbm, idx_v, rows_v, g_sems, s_sems):
        wid = lax.axis_index("s") * _NC + lax.axis_index("c")
        base = wid * n_per_w
        pltpu.sync_copy(idx_hbm.at[wid], idx_v)

        def start_gather(c, b):
            pltpu.async_copy(table_hbm.at[idx_v.at[c]], rows_v.at[b],
                             g_sems.at[b])

        def wait_gather(b):
            pltpu.make_async_copy(table_hbm.at[pl.ds(0, _CHUNK)],
                                  rows_v.at[b], g_sems.at[b]).wait()

        def start_store(c, b):
            pltpu.async_copy(rows_v.at[b],
                             out_hbm.at[pl.ds(base + c * _CHUNK, _CHUNK)],
                             s_sems.at[b])

        def wait_store(b):
            pltpu.make_async_copy(rows_v.at[b],
                                  out_hbm.at[pl.ds(base, _CHUNK)],
                                  s_sems.at[b]).wait()

        for c in range(_AHEAD):
            start_gather(c, c)

        def group(p, carry):
            for b in range(_NBUF):
                c = p * _NBUF + b
                t_buf = (b + _AHEAD) % _NBUF
                if b < _NBUF - _AHEAD:
                    @pl.when(p > 0)
                    def _():
                        wait_store(t_buf)
                    start_gather(c + _AHEAD, t_buf)
                else:
                    @pl.when(p < n_groups - 1)
                    def _():
                        wait_store(t_buf)
                        start_gather(c + _AHEAD, t_buf)
                wait_gather(b)
                start_store(c, b)
            return carry

        lax.fori_loop(0, n_groups, group, 0)
        for b in range(_NBUF):
            wait_store(b)

    return gather


def kernel(x, weight):
    flat = x.reshape(-1).astype(jnp.int32)
    idx3 = flat.reshape(_NW, -1, _CHUNK)
    out = _make_gather(flat.size, weight.shape[0])(idx3, weight)
    return out.reshape(x.shape + (weight.shape[-1],))
